# baseline (speedup 1.0000x reference)
import functools

import jax
import jax.numpy as jnp
from jax.experimental import pallas as pl
from jax.experimental.pallas import tpu as pltpu

_TARGET_BLOCK_BYTES = 6 * 1024 * 1024


def _charb_tile(x_ref, y_ref, out_ref, *, eps, rows_per_block):
    d = x_ref[...] - y_ref[...]
    err = jnp.sqrt(d * d + jnp.float32(eps))
    w = err.shape[-1]
    out_ref[0] = err.reshape(rows_per_block // 8, 8, w).sum(axis=0)


def _pick_block(leading, bytes_per_item):
    best = 1
    for b in range(1, leading + 1):
        if leading % b == 0 and b * bytes_per_item <= _TARGET_BLOCK_BYTES:
            best = b
    return best


def kernel(outputs, targets):
    eps = 1e-6
    shape = outputs.shape
    h, w = shape[-2], shape[-1]
    lead = 1
    for s in shape[:-2]:
        lead *= s
    n = lead * h * w

    x = outputs.reshape(lead, h, w)
    y = targets.reshape(lead, h, w)

    item_bytes = h * w * jnp.dtype(x.dtype).itemsize
    b = _pick_block(lead, item_bytes)
    num_tiles = lead // b

    body = functools.partial(_charb_tile, eps=eps, rows_per_block=b * h)

    partials = pl.pallas_call(
        body,
        out_shape=jax.ShapeDtypeStruct((num_tiles, 8, w), jnp.float32),
        grid=(num_tiles,),
        in_specs=[
            pl.BlockSpec((b, h, w), lambda t: (t, 0, 0)),
            pl.BlockSpec((b, h, w), lambda t: (t, 0, 0)),
        ],
        out_specs=pl.BlockSpec((1, 8, w), lambda t: (t, 0, 0)),
        compiler_params=pltpu.CompilerParams(
            dimension_semantics=("parallel",),
        ),
    )(x, y)

    return jnp.sum(partials) / jnp.float32(n)


# speedup vs baseline: 1.0293x; 1.0293x over previous
import functools

import jax
import jax.numpy as jnp
from jax.experimental import pallas as pl
from jax.experimental.pallas import tpu as pltpu

_TARGET_BLOCK_BYTES = 3 * 1024 * 1024


def _charb_tile(x_ref, y_ref, out_ref, *, eps, rows_per_block):
    d = x_ref[...] - y_ref[...]
    err = jnp.sqrt(d * d + jnp.float32(eps))
    w = err.shape[-1]
    out_ref[0] = err.reshape(rows_per_block // 8, 8, w).sum(axis=0)


def _pick_block(leading, bytes_per_item):
    best = 1
    for b in range(1, leading + 1):
        if leading % b == 0 and b * bytes_per_item <= _TARGET_BLOCK_BYTES:
            best = b
    return best


def kernel(outputs, targets):
    eps = 1e-6
    shape = outputs.shape
    h, w = shape[-2], shape[-1]
    lead = 1
    for s in shape[:-2]:
        lead *= s
    n = lead * h * w

    x = outputs.reshape(lead, h, w)
    y = targets.reshape(lead, h, w)

    item_bytes = h * w * jnp.dtype(x.dtype).itemsize
    b = _pick_block(lead, item_bytes)
    num_tiles = lead // b

    body = functools.partial(_charb_tile, eps=eps, rows_per_block=b * h)

    partials = pl.pallas_call(
        body,
        out_shape=jax.ShapeDtypeStruct((num_tiles, 8, w), jnp.float32),
        grid=(num_tiles,),
        in_specs=[
            pl.BlockSpec((b, h, w), lambda t: (t, 0, 0)),
            pl.BlockSpec((b, h, w), lambda t: (t, 0, 0)),
        ],
        out_specs=pl.BlockSpec((1, 8, w), lambda t: (t, 0, 0)),
        compiler_params=pltpu.CompilerParams(
            dimension_semantics=("parallel",),
        ),
    )(x, y)

    return jnp.sum(partials) / jnp.float32(n)


# speedup vs baseline: 1.0319x; 1.0026x over previous
import functools

import jax
import jax.numpy as jnp
from jax.experimental import pallas as pl
from jax.experimental.pallas import tpu as pltpu

_TARGET_BLOCK_BYTES = 3 * 1024 * 1024


def _charb_tile(x_ref, y_ref, out_ref, *, eps, rows_per_block):
    d = x_ref[...] - y_ref[...]
    err = jnp.sqrt(d * d + jnp.float32(eps))
    w = err.shape[-1]
    out_ref[0] = err.reshape(rows_per_block // 8, 8, w).sum(axis=0)


def _pick_block(leading, bytes_per_item):
    best = 1
    for b in range(1, leading + 1):
        if leading % b == 0 and b * bytes_per_item <= _TARGET_BLOCK_BYTES:
            best = b
    return best


def kernel(outputs, targets):
    eps = 1e-6
    shape = outputs.shape
    h, w = shape[-2], shape[-1]
    lead = 1
    for s in shape[:-2]:
        lead *= s
    n = lead * h * w

    x = outputs.reshape(lead, h, w)
    y = targets.reshape(lead, h, w)

    item_bytes = h * w * jnp.dtype(x.dtype).itemsize
    b = _pick_block(lead, item_bytes)
    num_tiles = lead // b

    body = functools.partial(_charb_tile, eps=eps, rows_per_block=b * h)

    partials = pl.pallas_call(
        body,
        out_shape=jax.ShapeDtypeStruct((num_tiles, 8, w), jnp.float32),
        grid=(num_tiles,),
        in_specs=[
            pl.BlockSpec((b, h, w), lambda t: (t, 0, 0)),
            pl.BlockSpec((b, h, w), lambda t: (t, 0, 0)),
        ],
        out_specs=pl.BlockSpec((1, 8, w), lambda t: (t, 0, 0)),
        compiler_params=pltpu.CompilerParams(
            dimension_semantics=("arbitrary",),
        ),
    )(x, y)

    return jnp.sum(partials) / jnp.float32(n)


# speedup vs baseline: 1.2137x; 1.1762x over previous
import functools

import jax
import jax.numpy as jnp
from jax.experimental import pallas as pl
from jax.experimental.pallas import tpu as pltpu

_TARGET_BLOCK_BYTES = 3 * 1024 * 1024


def _charb_step(x_ref, y_ref, out_ref, acc_ref, *, eps, rows_per_block,
                num_tiles, inv_n):
    t = pl.program_id(0)
    d = x_ref[...] - y_ref[...]
    err = jnp.sqrt(d * d + jnp.float32(eps))
    w = err.shape[-1]
    folded = err.reshape(rows_per_block // 8, 8, w).sum(axis=0)

    @pl.when(t == 0)
    def _():
        acc_ref[...] = folded

    @pl.when(t > 0)
    def _():
        acc_ref[...] += folded

    @pl.when(t == num_tiles - 1)
    def _():
        out_ref[0, 0] = jnp.sum(acc_ref[...]) * jnp.float32(inv_n)


def _pick_block(leading, bytes_per_item):
    best = 1
    for b in range(1, leading + 1):
        if leading % b == 0 and b * bytes_per_item <= _TARGET_BLOCK_BYTES:
            best = b
    return best


def kernel(outputs, targets):
    eps = 1e-6
    shape = outputs.shape
    h, w = shape[-2], shape[-1]
    lead = 1
    for s in shape[:-2]:
        lead *= s
    n = lead * h * w

    x = outputs.reshape(lead, h, w)
    y = targets.reshape(lead, h, w)

    item_bytes = h * w * jnp.dtype(x.dtype).itemsize
    b = _pick_block(lead, item_bytes)
    num_tiles = lead // b

    body = functools.partial(_charb_step, eps=eps, rows_per_block=b * h,
                             num_tiles=num_tiles, inv_n=1.0 / n)

    loss = pl.pallas_call(
        body,
        out_shape=jax.ShapeDtypeStruct((1, 1), jnp.float32),
        grid=(num_tiles,),
        in_specs=[
            pl.BlockSpec((b, h, w), lambda t: (t, 0, 0)),
            pl.BlockSpec((b, h, w), lambda t: (t, 0, 0)),
        ],
        out_specs=pl.BlockSpec(memory_space=pltpu.SMEM),
        scratch_shapes=[pltpu.VMEM((8, w), jnp.float32)],
        compiler_params=pltpu.CompilerParams(
            dimension_semantics=("arbitrary",),
        ),
    )(x, y)

    return loss.reshape(())


# speedup vs baseline: 1.3785x; 1.1358x over previous
import functools

import jax
import jax.numpy as jnp
from jax.experimental import pallas as pl
from jax.experimental.pallas import tpu as pltpu

_TARGET_BLOCK_BYTES = 3 * 1024 * 1024


def _charb_step(x_ref, y_ref, out_ref, acc_ref, *, eps, rows_per_block,
                num_tiles, inv_n):
    t = pl.program_id(0)
    d = x_ref[...] - y_ref[...]
    v = d * d + jnp.float32(eps)
    err = v * jax.lax.rsqrt(v)
    w = err.shape[-1]
    folded = err.reshape(rows_per_block // 8, 8, w).sum(axis=0)

    @pl.when(t == 0)
    def _():
        acc_ref[...] = folded

    @pl.when(t > 0)
    def _():
        acc_ref[...] += folded

    @pl.when(t == num_tiles - 1)
    def _():
        out_ref[0, 0] = jnp.sum(acc_ref[...]) * jnp.float32(inv_n)


def _pick_block(leading, bytes_per_item):
    best = 1
    for b in range(1, leading + 1):
        if leading % b == 0 and b * bytes_per_item <= _TARGET_BLOCK_BYTES:
            best = b
    return best


def kernel(outputs, targets):
    eps = 1e-6
    shape = outputs.shape
    h, w = shape[-2], shape[-1]
    lead = 1
    for s in shape[:-2]:
        lead *= s
    n = lead * h * w

    x = outputs.reshape(lead, h, w)
    y = targets.reshape(lead, h, w)

    item_bytes = h * w * jnp.dtype(x.dtype).itemsize
    b = _pick_block(lead, item_bytes)
    num_tiles = lead // b

    body = functools.partial(_charb_step, eps=eps, rows_per_block=b * h,
                             num_tiles=num_tiles, inv_n=1.0 / n)

    loss = pl.pallas_call(
        body,
        out_shape=jax.ShapeDtypeStruct((1, 1), jnp.float32),
        grid=(num_tiles,),
        in_specs=[
            pl.BlockSpec((b, h, w), lambda t: (t, 0, 0)),
            pl.BlockSpec((b, h, w), lambda t: (t, 0, 0)),
        ],
        out_specs=pl.BlockSpec(memory_space=pltpu.SMEM),
        scratch_shapes=[pltpu.VMEM((8, w), jnp.float32)],
        compiler_params=pltpu.CompilerParams(
            dimension_semantics=("arbitrary",),
        ),
    )(x, y)

    return loss.reshape(())
